# baseline (speedup 1.0000x reference)
import functools
import math

import jax
import jax.numpy as jnp
from jax import lax
from jax.experimental import pallas as pl
from jax.experimental.pallas import tpu as pltpu
from jax.experimental.pallas import tpu_sc as plsc

N = 10000
E = 320000
H = 128
NE = 4
STEPS = 2

NW = 32
C = 64
NCH = 160
NBUF = 5
E_PAD = NW * NCH * C
NPAD = 10112
ROWS_PER_TILE = NPAD // 16
OUT_ROWS_PER_TILE = 624

BN = 2000


def _th_body(h_ref, wt_ref, b_ref, out_ref):
    hv = h_ref[...]
    wt = wt_ref[...]
    bv = b_ref[...]
    for e in range(NE):
        out_ref[e, :, :] = (
            jnp.dot(hv, wt[e], preferred_element_type=jnp.float32)
            + bv[e, :][None, :]
        )


def _th_call(h, wt, b):
    return pl.pallas_call(
        _th_body,
        grid=(N // BN,),
        in_specs=[
            pl.BlockSpec((BN, H), lambda i: (i, 0)),
            pl.BlockSpec((NE, H, H), lambda i: (0, 0, 0)),
            pl.BlockSpec((NE, H), lambda i: (0, 0)),
        ],
        out_specs=pl.BlockSpec((NE, BN, H), lambda i: (0, i, 0)),
        out_shape=jax.ShapeDtypeStruct((NE, N, H), jnp.float32),
    )(h, wt, b)


_sc_mesh = plsc.VectorSubcoreMesh(core_axis_name="c", subcore_axis_name="s")


@functools.partial(
    pl.kernel,
    out_type=jax.ShapeDtypeStruct((2, N, H), jnp.float32),
    mesh=_sc_mesh,
    scratch_types=[
        pltpu.VMEM((NCH // 8, C), jnp.int32),
        pltpu.VMEM((NCH // 8, C), jnp.int32),
        pltpu.VMEM((C, H), jnp.float32),
        pltpu.VMEM((C, H), jnp.float32),
        pltpu.VMEM((C, H), jnp.float32),
        pltpu.VMEM((C, H), jnp.float32),
        pltpu.VMEM((C, H), jnp.float32),
        pltpu.VMEM_SHARED((NPAD, H), jnp.float32),
        pltpu.SemaphoreType.DMA,
    ],
)
def _sc_scatter(tab_hbm, comb_hbm, dst_hbm, zeros_hbm, out_hbm,
                comb_v, dst_v, b0, b1, b2, b3, b4, acc, gsem):
    bufs = (b0, b1, b2, b3, b4)
    cid = lax.axis_index("c")
    sid = lax.axis_index("s")
    wid = sid * 2 + cid

    buf0 = bufs[0]
    pltpu.sync_copy(zeros_hbm, buf0)
    zbase = sid * ROWS_PER_TILE
    nfull = ROWS_PER_TILE // C
    rem = ROWS_PER_TILE % C
    for k in range(nfull):
        pltpu.sync_copy(buf0, acc.at[pl.ds(zbase + k * C, C)])
    if rem:
        pltpu.sync_copy(buf0.at[pl.ds(0, rem)],
                        acc.at[pl.ds(zbase + nfull * C, rem)])
    plsc.subcore_barrier()

    NCHH = NCH // 8

    def gstart(j, b):
        pltpu.async_copy(tab_hbm.at[comb_v.at[j]], b, gsem)

    def gwait(j, b):
        pltpu.make_async_copy(tab_hbm.at[comb_v.at[j]], b, gsem).wait()

    for hf in range(8):
        pltpu.sync_copy(comb_hbm.at[wid, hf], comb_v)
        pltpu.sync_copy(dst_hbm.at[wid, hf], dst_v)
        for b in range(NBUF):
            gstart(b, bufs[b])

        def body(i, carry):
            for b in range(NBUF):
                j = NBUF * i + b
                gwait(j, bufs[b])
                pltpu.sync_copy(bufs[b], acc.at[dst_v.at[j]], add=True)

                @pl.when(j + NBUF < NCHH)
                def _():
                    gstart(j + NBUF, bufs[b])

            return carry

        lax.fori_loop(0, NCHH // NBUF, body, 0)
    plsc.subcore_barrier()

    obase = sid * OUT_ROWS_PER_TILE
    pltpu.sync_copy(acc.at[pl.ds(obase, OUT_ROWS_PER_TILE)],
                    out_hbm.at[cid, pl.ds(obase, OUT_ROWS_PER_TILE)])

    tail = N - 16 * OUT_ROWS_PER_TILE
    @pl.when(sid == 15)
    def _():
        pltpu.sync_copy(acc.at[pl.ds(16 * OUT_ROWS_PER_TILE, tail)],
                        out_hbm.at[cid, pl.ds(16 * OUT_ROWS_PER_TILE, tail)])


def _gru_core(parts_ref, h_ref, wih_ref, whh_ref, bih_ref, bhh_ref):
    a = parts_ref[0, :, :] + parts_ref[1, :, :]
    hv = h_ref[...]
    gi = jnp.dot(a, wih_ref[...], preferred_element_type=jnp.float32) + bih_ref[...]
    gh = jnp.dot(hv, whh_ref[...], preferred_element_type=jnp.float32) + bhh_ref[...]
    r = 1.0 / (1.0 + jnp.exp(-(gi[:, :H] + gh[:, :H])))
    z = 1.0 / (1.0 + jnp.exp(-(gi[:, H:2 * H] + gh[:, H:2 * H])))
    n = jnp.tanh(gi[:, 2 * H:] + r * gh[:, 2 * H:])
    return (1.0 - z) * n + z * hv


def _gru_tab_body(parts_ref, h_ref, wih_ref, whh_ref, bih_ref, bhh_ref,
                  wt_ref, b_ref, tab_ref, hn_ref):
    hn = _gru_core(parts_ref, h_ref, wih_ref, whh_ref, bih_ref, bhh_ref)
    hn_ref[...] = hn
    wt = wt_ref[...]
    bv = b_ref[...]
    for e in range(NE):
        tab_ref[e, :, :] = (
            jnp.dot(hn, wt[e], preferred_element_type=jnp.float32)
            + bv[e, :][None, :]
        )


def _gru_tab_call(parts, h, wih_t, whh_t, bih, bhh, wt, b):
    return pl.pallas_call(
        _gru_tab_body,
        grid=(N // BN,),
        in_specs=[
            pl.BlockSpec((2, BN, H), lambda i: (0, i, 0)),
            pl.BlockSpec((BN, H), lambda i: (i, 0)),
            pl.BlockSpec((H, 3 * H), lambda i: (0, 0)),
            pl.BlockSpec((H, 3 * H), lambda i: (0, 0)),
            pl.BlockSpec((1, 3 * H), lambda i: (0, 0)),
            pl.BlockSpec((1, 3 * H), lambda i: (0, 0)),
            pl.BlockSpec((NE, H, H), lambda i: (0, 0, 0)),
            pl.BlockSpec((NE, H), lambda i: (0, 0)),
        ],
        out_specs=[
            pl.BlockSpec((NE, BN, H), lambda i: (0, i, 0)),
            pl.BlockSpec((BN, H), lambda i: (i, 0)),
        ],
        out_shape=[
            jax.ShapeDtypeStruct((NE, N, H), jnp.float32),
            jax.ShapeDtypeStruct((N, H), jnp.float32),
        ],
    )(parts, h, wih_t, whh_t, bih, bhh, wt, b)


def _gru_fin_body(parts_ref, h_ref, wih_ref, whh_ref, bih_ref, bhh_ref,
                  gamma_ref, beta_ref, wg_ref, h2_ref, gate_ref):
    hn = _gru_core(parts_ref, h_ref, wih_ref, whh_ref, bih_ref, bhh_ref)
    h2 = jnp.where(hn > 0.0, hn, jnp.exp(hn) - 1.0)
    h2 = h2 * (gamma_ref[...] * _BN_SCALE) + beta_ref[...]
    h2_ref[...] = h2
    gate_ref[...] = jnp.sum(h2 * wg_ref[...], axis=1, keepdims=True)


def _gru_fin_call(parts, h, wih_t, whh_t, bih, bhh, gamma, beta, wg):
    return pl.pallas_call(
        _gru_fin_body,
        grid=(N // BN,),
        in_specs=[
            pl.BlockSpec((2, BN, H), lambda i: (0, i, 0)),
            pl.BlockSpec((BN, H), lambda i: (i, 0)),
            pl.BlockSpec((H, 3 * H), lambda i: (0, 0)),
            pl.BlockSpec((H, 3 * H), lambda i: (0, 0)),
            pl.BlockSpec((1, 3 * H), lambda i: (0, 0)),
            pl.BlockSpec((1, 3 * H), lambda i: (0, 0)),
            pl.BlockSpec((1, H), lambda i: (0, 0)),
            pl.BlockSpec((1, H), lambda i: (0, 0)),
            pl.BlockSpec((1, H), lambda i: (0, 0)),
        ],
        out_specs=[
            pl.BlockSpec((BN, H), lambda i: (i, 0)),
            pl.BlockSpec((BN, 1), lambda i: (i, 0)),
        ],
        out_shape=[
            jax.ShapeDtypeStruct((N, H), jnp.float32),
            jax.ShapeDtypeStruct((N, 1), jnp.float32),
        ],
    )(parts, h, wih_t, whh_t, bih, bhh, gamma, beta, wg)


_BN_SCALE = 1.0 / math.sqrt(1.0 + 1e-5)


def _pool_body(h2_ref, gate_ref, w1_ref, b1_ref, w2_ref, b2_ref, out_ref):
    h2 = h2_ref[...]
    gate = gate_ref[...]
    m = jnp.max(gate)
    ex = jnp.exp(gate - m)
    alpha = ex / jnp.sum(ex)
    hg = jnp.sum(alpha * h2, axis=0, keepdims=True)
    hgb = jnp.broadcast_to(hg, (8, H))
    t1 = jnp.dot(hgb, w1_ref[...], preferred_element_type=jnp.float32) + b1_ref[...]
    t1 = jnp.maximum(t1, 0.0)
    t2 = jnp.dot(t1, w2_ref[...], preferred_element_type=jnp.float32) + b2_ref[...]
    out_ref[...] = t2[0:1, :]


def _pool_call(h2, gate, w1_t, b1, w2_t, b2):
    return pl.pallas_call(
        _pool_body,
        out_shape=jax.ShapeDtypeStruct((1, 10), jnp.float32),
    )(h2, gate, w1_t, b1, w2_t, b2)


def kernel(x, edge_index, edge_type, W_etype, b_etype, W_ih, W_hh, b_ih, b_hh,
           bn_gamma, bn_beta, Wg, bg, W1, b1, W2, b2):
    src = edge_index[0]
    dst = edge_index[1]
    et = edge_type.astype(jnp.int32)

    pad = E_PAD - E
    if pad >= 0:
        ar = jnp.arange(pad, dtype=jnp.int32)
        comb = jnp.concatenate([et * N + src, (ar * 37) % (NE * N)])
        dstp = jnp.concatenate([dst, N + (ar % (NPAD - N))])
    else:
        comb = (et * N + src)[:E_PAD]
        dstp = dst[:E_PAD]
    comb3 = comb.reshape(NW, 8, NCH // 8, C)
    dst3 = dstp.reshape(NW, 8, NCH // 8, C)
    zeros = jnp.zeros((C, H), jnp.float32)

    wt = jnp.swapaxes(W_etype, 1, 2)
    wih_t = W_ih.T
    whh_t = W_hh.T
    bih = b_ih.reshape(1, 3 * H)
    bhh = b_hh.reshape(1, 3 * H)

    tab = _th_call(x, wt, b_etype)
    parts = _sc_scatter(tab.reshape(NE * N, H), comb3, dst3, zeros)
    tab2, h1 = _gru_tab_call(parts, x, wih_t, whh_t, bih, bhh, wt, b_etype)
    parts2 = _sc_scatter(tab2.reshape(NE * N, H), comb3, dst3, zeros)
    h2, gate = _gru_fin_call(parts2, h1, wih_t, whh_t, bih, bhh,
                             bn_gamma.reshape(1, H), bn_beta.reshape(1, H),
                             Wg.reshape(1, H))
    return _pool_call(h2, gate, W1.T, b1.reshape(1, H // 2), W2.T,
                      b2.reshape(1, 10))


# speedup vs baseline: 1.0759x; 1.0759x over previous
import functools
import math

import jax
import jax.numpy as jnp
from jax import lax
from jax.experimental import pallas as pl
from jax.experimental.pallas import tpu as pltpu
from jax.experimental.pallas import tpu_sc as plsc

N = 10000
E = 320000
H = 128
NE = 4
STEPS = 2

NW = 32
C = 64
NCH = 160
NBUF = 4
E_PAD = NW * NCH * C
NPAD = 10112
ROWS_PER_TILE = NPAD // 16
OUT_ROWS_PER_TILE = 624

BN = 2000


def _th_body(h_ref, wt_ref, b_ref, out_ref):
    hv = h_ref[...]
    wt = wt_ref[...]
    bv = b_ref[...]
    for e in range(NE):
        out_ref[e, :, :] = (
            jnp.dot(hv, wt[e], preferred_element_type=jnp.float32)
            + bv[e, :][None, :]
        )


def _th_call(h, wt, b):
    return pl.pallas_call(
        _th_body,
        grid=(N // BN,),
        in_specs=[
            pl.BlockSpec((BN, H), lambda i: (i, 0)),
            pl.BlockSpec((NE, H, H), lambda i: (0, 0, 0)),
            pl.BlockSpec((NE, H), lambda i: (0, 0)),
        ],
        out_specs=pl.BlockSpec((NE, BN, H), lambda i: (0, i, 0)),
        out_shape=jax.ShapeDtypeStruct((NE, N, H), jnp.float32),
    )(h, wt, b)


_sc_mesh = plsc.VectorSubcoreMesh(core_axis_name="c", subcore_axis_name="s")


@functools.partial(
    pl.kernel,
    out_type=jax.ShapeDtypeStruct((2, N, H), jnp.float32),
    mesh=_sc_mesh,
    scratch_types=[
        pltpu.VMEM((NCH // 4, C), jnp.int32),
        pltpu.VMEM((NCH // 4, C), jnp.int32),
        pltpu.VMEM((C, H), jnp.float32),
        pltpu.VMEM((C, H), jnp.float32),
        pltpu.VMEM((C, H), jnp.float32),
        pltpu.VMEM((C, H), jnp.float32),
        pltpu.VMEM_SHARED((NPAD, H), jnp.float32),
        pltpu.SemaphoreType.DMA,
    ],
)
def _sc_scatter(tab_hbm, comb_hbm, dst_hbm, zeros_hbm, out_hbm,
                comb_v, dst_v, b0, b1, b2, b3, acc, gsem):
    bufs = (b0, b1, b2, b3)
    cid = lax.axis_index("c")
    sid = lax.axis_index("s")
    wid = sid * 2 + cid

    buf0 = bufs[0]
    pltpu.sync_copy(zeros_hbm, buf0)
    zbase = sid * ROWS_PER_TILE
    nfull = ROWS_PER_TILE // C
    rem = ROWS_PER_TILE % C
    for k in range(nfull):
        pltpu.sync_copy(buf0, acc.at[pl.ds(zbase + k * C, C)])
    if rem:
        pltpu.sync_copy(buf0.at[pl.ds(0, rem)],
                        acc.at[pl.ds(zbase + nfull * C, rem)])
    plsc.subcore_barrier()

    NCHH = NCH // 4

    def gstart(j, b):
        pltpu.async_copy(tab_hbm.at[comb_v.at[j]], b, gsem)

    def gwait(j, b):
        pltpu.make_async_copy(tab_hbm.at[comb_v.at[j]], b, gsem).wait()

    for hf in range(4):
        pltpu.sync_copy(comb_hbm.at[wid, hf], comb_v)
        pltpu.sync_copy(dst_hbm.at[wid, hf], dst_v)
        for b in range(NBUF):
            gstart(b, bufs[b])

        def body(i, carry):
            for b in range(NBUF):
                j = NBUF * i + b
                gwait(j, bufs[b])
                pltpu.sync_copy(bufs[b], acc.at[dst_v.at[j]], add=True)

                @pl.when(j + NBUF < NCHH)
                def _():
                    gstart(j + NBUF, bufs[b])

            return carry

        lax.fori_loop(0, NCHH // NBUF, body, 0)
    plsc.subcore_barrier()

    obase = sid * OUT_ROWS_PER_TILE
    pltpu.sync_copy(acc.at[pl.ds(obase, OUT_ROWS_PER_TILE)],
                    out_hbm.at[cid, pl.ds(obase, OUT_ROWS_PER_TILE)])

    tail = N - 16 * OUT_ROWS_PER_TILE
    @pl.when(sid == 15)
    def _():
        pltpu.sync_copy(acc.at[pl.ds(16 * OUT_ROWS_PER_TILE, tail)],
                        out_hbm.at[cid, pl.ds(16 * OUT_ROWS_PER_TILE, tail)])


def _gru_core(parts_ref, h_ref, wih_ref, whh_ref, bih_ref, bhh_ref):
    a = parts_ref[0, :, :] + parts_ref[1, :, :]
    hv = h_ref[...]
    gi = jnp.dot(a, wih_ref[...], preferred_element_type=jnp.float32) + bih_ref[...]
    gh = jnp.dot(hv, whh_ref[...], preferred_element_type=jnp.float32) + bhh_ref[...]
    r = 1.0 / (1.0 + jnp.exp(-(gi[:, :H] + gh[:, :H])))
    z = 1.0 / (1.0 + jnp.exp(-(gi[:, H:2 * H] + gh[:, H:2 * H])))
    n = jnp.tanh(gi[:, 2 * H:] + r * gh[:, 2 * H:])
    return (1.0 - z) * n + z * hv


def _gru_tab_body(parts_ref, h_ref, wih_ref, whh_ref, bih_ref, bhh_ref,
                  wt_ref, b_ref, tab_ref, hn_ref):
    hn = _gru_core(parts_ref, h_ref, wih_ref, whh_ref, bih_ref, bhh_ref)
    hn_ref[...] = hn
    wt = wt_ref[...]
    bv = b_ref[...]
    for e in range(NE):
        tab_ref[e, :, :] = (
            jnp.dot(hn, wt[e], preferred_element_type=jnp.float32)
            + bv[e, :][None, :]
        )


def _gru_tab_call(parts, h, wih_t, whh_t, bih, bhh, wt, b):
    return pl.pallas_call(
        _gru_tab_body,
        grid=(N // BN,),
        in_specs=[
            pl.BlockSpec((2, BN, H), lambda i: (0, i, 0)),
            pl.BlockSpec((BN, H), lambda i: (i, 0)),
            pl.BlockSpec((H, 3 * H), lambda i: (0, 0)),
            pl.BlockSpec((H, 3 * H), lambda i: (0, 0)),
            pl.BlockSpec((1, 3 * H), lambda i: (0, 0)),
            pl.BlockSpec((1, 3 * H), lambda i: (0, 0)),
            pl.BlockSpec((NE, H, H), lambda i: (0, 0, 0)),
            pl.BlockSpec((NE, H), lambda i: (0, 0)),
        ],
        out_specs=[
            pl.BlockSpec((NE, BN, H), lambda i: (0, i, 0)),
            pl.BlockSpec((BN, H), lambda i: (i, 0)),
        ],
        out_shape=[
            jax.ShapeDtypeStruct((NE, N, H), jnp.float32),
            jax.ShapeDtypeStruct((N, H), jnp.float32),
        ],
    )(parts, h, wih_t, whh_t, bih, bhh, wt, b)


def _gru_fin_body(parts_ref, h_ref, wih_ref, whh_ref, bih_ref, bhh_ref,
                  gamma_ref, beta_ref, wg_ref, h2_ref, gate_ref):
    hn = _gru_core(parts_ref, h_ref, wih_ref, whh_ref, bih_ref, bhh_ref)
    h2 = jnp.where(hn > 0.0, hn, jnp.exp(hn) - 1.0)
    h2 = h2 * (gamma_ref[...] * _BN_SCALE) + beta_ref[...]
    h2_ref[...] = h2
    gate_ref[...] = jnp.sum(h2 * wg_ref[...], axis=1, keepdims=True)


def _gru_fin_call(parts, h, wih_t, whh_t, bih, bhh, gamma, beta, wg):
    return pl.pallas_call(
        _gru_fin_body,
        grid=(N // BN,),
        in_specs=[
            pl.BlockSpec((2, BN, H), lambda i: (0, i, 0)),
            pl.BlockSpec((BN, H), lambda i: (i, 0)),
            pl.BlockSpec((H, 3 * H), lambda i: (0, 0)),
            pl.BlockSpec((H, 3 * H), lambda i: (0, 0)),
            pl.BlockSpec((1, 3 * H), lambda i: (0, 0)),
            pl.BlockSpec((1, 3 * H), lambda i: (0, 0)),
            pl.BlockSpec((1, H), lambda i: (0, 0)),
            pl.BlockSpec((1, H), lambda i: (0, 0)),
            pl.BlockSpec((1, H), lambda i: (0, 0)),
        ],
        out_specs=[
            pl.BlockSpec((BN, H), lambda i: (i, 0)),
            pl.BlockSpec((BN, 1), lambda i: (i, 0)),
        ],
        out_shape=[
            jax.ShapeDtypeStruct((N, H), jnp.float32),
            jax.ShapeDtypeStruct((N, 1), jnp.float32),
        ],
    )(parts, h, wih_t, whh_t, bih, bhh, gamma, beta, wg)


_BN_SCALE = 1.0 / math.sqrt(1.0 + 1e-5)


def _pool_body(h2_ref, gate_ref, w1_ref, b1_ref, w2_ref, b2_ref, out_ref):
    h2 = h2_ref[...]
    gate = gate_ref[...]
    m = jnp.max(gate)
    ex = jnp.exp(gate - m)
    alpha = ex / jnp.sum(ex)
    hg = jnp.sum(alpha * h2, axis=0, keepdims=True)
    hgb = jnp.broadcast_to(hg, (8, H))
    t1 = jnp.dot(hgb, w1_ref[...], preferred_element_type=jnp.float32) + b1_ref[...]
    t1 = jnp.maximum(t1, 0.0)
    t2 = jnp.dot(t1, w2_ref[...], preferred_element_type=jnp.float32) + b2_ref[...]
    out_ref[...] = t2[0:1, :]


def _pool_call(h2, gate, w1_t, b1, w2_t, b2):
    return pl.pallas_call(
        _pool_body,
        out_shape=jax.ShapeDtypeStruct((1, 10), jnp.float32),
    )(h2, gate, w1_t, b1, w2_t, b2)


def kernel(x, edge_index, edge_type, W_etype, b_etype, W_ih, W_hh, b_ih, b_hh,
           bn_gamma, bn_beta, Wg, bg, W1, b1, W2, b2):
    src = edge_index[0]
    dst = edge_index[1]
    et = edge_type.astype(jnp.int32)

    pad = E_PAD - E
    if pad >= 0:
        ar = jnp.arange(pad, dtype=jnp.int32)
        comb = jnp.concatenate([et * N + src, (ar * 37) % (NE * N)])
        dstp = jnp.concatenate([dst, N + (ar % (NPAD - N))])
    else:
        comb = (et * N + src)[:E_PAD]
        dstp = dst[:E_PAD]
    comb3 = comb.reshape(NW, 4, NCH // 4, C)
    dst3 = dstp.reshape(NW, 4, NCH // 4, C)
    zeros = jnp.zeros((C, H), jnp.float32)

    wt = jnp.swapaxes(W_etype, 1, 2)
    wih_t = W_ih.T
    whh_t = W_hh.T
    bih = b_ih.reshape(1, 3 * H)
    bhh = b_hh.reshape(1, 3 * H)

    tab = _th_call(x, wt, b_etype)
    parts = _sc_scatter(tab.reshape(NE * N, H), comb3, dst3, zeros)
    tab2, h1 = _gru_tab_call(parts, x, wih_t, whh_t, bih, bhh, wt, b_etype)
    parts2 = _sc_scatter(tab2.reshape(NE * N, H), comb3, dst3, zeros)
    h2, gate = _gru_fin_call(parts2, h1, wih_t, whh_t, bih, bhh,
                             bn_gamma.reshape(1, H), bn_beta.reshape(1, H),
                             Wg.reshape(1, H))
    return _pool_call(h2, gate, W1.T, b1.reshape(1, H // 2), W2.T,
                      b2.reshape(1, 10))
